# baseline (speedup 1.0000x reference)
import functools

import jax
import jax.numpy as jnp
from jax import lax
from jax.experimental import pallas as pl
from jax.experimental.pallas import tpu as pltpu
from jax.experimental.pallas import tpu_sc as plsc


def _make_gather(B, V, D):
    info = plsc.get_sparse_core_info()
    nc, ns = info.num_cores, info.num_subcores
    nw = nc * ns
    b_per_w = B // nw
    CH = 1600
    n_ch = b_per_w // CH

    mesh = plsc.VectorSubcoreMesh(core_axis_name="c", subcore_axis_name="s")

    @functools.partial(
        pl.kernel,
        mesh=mesh,
        compiler_params=pltpu.CompilerParams(use_tc_tiling_on_sc=False),
        out_type=jax.ShapeDtypeStruct((B, D), jnp.float32),
        scratch_types=[
            pltpu.VMEM((b_per_w,), jnp.int32),
            pltpu.VMEM((2, CH, D), jnp.float32),
            pltpu.SemaphoreType.DMA,
        ],
    )
    def gather_kernel(idx_hbm, table_hbm, out_hbm, idx_v, rows_v, gsem):
        wid = lax.axis_index("s") * nc + lax.axis_index("c")
        base = wid * b_per_w
        pltpu.sync_copy(idx_hbm.at[pl.ds(base, b_per_w)], idx_v)
        for c in range(n_ch):
            buf = rows_v.at[c % 2]
            pltpu.async_copy(
                table_hbm.at[idx_v.at[pl.ds(c * CH, CH)]], buf, gsem
            ).wait()
            pltpu.sync_copy(buf, out_hbm.at[pl.ds(base + c * CH, CH)])

    return gather_kernel


def kernel(x, table):
    B0, S = x.shape
    V, D = table.shape
    B = B0 * S
    out = _make_gather(B, V, D)(x.reshape(-1), table)
    return out.reshape(B0, S, D)


# speedup vs baseline: 1.0009x; 1.0009x over previous
import functools

import jax
import jax.numpy as jnp
from jax import lax
from jax.experimental import pallas as pl
from jax.experimental.pallas import tpu as pltpu
from jax.experimental.pallas import tpu_sc as plsc

_NBUF = 4
_LOOKAHEAD = 3


def _make_gather(B, V, D):
    info = plsc.get_sparse_core_info()
    nc, ns = info.num_cores, info.num_subcores
    nw = nc * ns
    b_per_w = B // nw
    CH = 800
    n_ch = b_per_w // CH

    mesh = plsc.VectorSubcoreMesh(core_axis_name="c", subcore_axis_name="s")

    @functools.partial(
        pl.kernel,
        mesh=mesh,
        compiler_params=pltpu.CompilerParams(use_tc_tiling_on_sc=False),
        out_type=jax.ShapeDtypeStruct((B, D), jnp.float32),
        scratch_types=[
            pltpu.VMEM((b_per_w,), jnp.int32),
            pltpu.VMEM((_NBUF, CH, D), jnp.float32),
            [pltpu.SemaphoreType.DMA] * _NBUF,
            [pltpu.SemaphoreType.DMA] * _NBUF,
        ],
    )
    def gather_kernel(idx_hbm, table_hbm, out_hbm, idx_v, rows_v, gsems, osems):
        wid = lax.axis_index("s") * nc + lax.axis_index("c")
        base = wid * b_per_w
        pltpu.sync_copy(idx_hbm.at[pl.ds(base, b_per_w)], idx_v)

        def start_gather(c):
            return pltpu.async_copy(
                table_hbm.at[idx_v.at[pl.ds(c * CH, CH)]],
                rows_v.at[c % _NBUF],
                gsems[c % _NBUF],
            )

        def start_store(c):
            return pltpu.async_copy(
                rows_v.at[c % _NBUF],
                out_hbm.at[pl.ds(base + c * CH, CH)],
                osems[c % _NBUF],
            )

        for c in range(min(_LOOKAHEAD, n_ch)):
            start_gather(c)
        stores = {}
        for c in range(n_ch):
            pltpu.make_async_copy(
                table_hbm.at[idx_v.at[pl.ds(c * CH, CH)]],
                rows_v.at[c % _NBUF],
                gsems[c % _NBUF],
            ).wait()
            stores[c] = start_store(c)
            nxt = c + _LOOKAHEAD
            if nxt < n_ch:
                prev = nxt - _NBUF
                if prev >= 0:
                    stores.pop(prev).wait()
                start_gather(nxt)
        for c in sorted(stores):
            stores[c].wait()

    return gather_kernel


def kernel(x, table):
    B0, S = x.shape
    V, D = table.shape
    B = B0 * S
    out = _make_gather(B, V, D)(x.reshape(-1), table)
    return out.reshape(B0, S, D)


# speedup vs baseline: 1.0352x; 1.0342x over previous
import functools

import jax
import jax.numpy as jnp
from jax import lax
from jax.experimental import pallas as pl
from jax.experimental.pallas import tpu as pltpu
from jax.experimental.pallas import tpu_sc as plsc


def _make_gather(S, B0, V, D):
    info = plsc.get_sparse_core_info()
    nc, ns = info.num_cores, info.num_subcores
    nw = nc * ns
    bw = B0 // nw
    L = info.num_lanes
    ng = bw // L
    pack = 128 // D

    mesh = plsc.VectorSubcoreMesh(core_axis_name="c", subcore_axis_name="s")

    @functools.partial(
        pl.kernel,
        mesh=mesh,
        compiler_params=pltpu.CompilerParams(
            use_tc_tiling_on_sc=True, needs_layout_passes=False
        ),
        out_type=jax.ShapeDtypeStruct((S, D, B0), jnp.float32),
        scratch_types=[
            pltpu.VMEM((56, bw), jnp.int32),
            pltpu.VMEM((2, bw), jnp.int32),
            pltpu.VMEM((2, bw), jnp.int32),
            pltpu.VMEM((2, bw, 128), jnp.float32),
            pltpu.VMEM((2, D, bw), jnp.float32),
            [pltpu.SemaphoreType.DMA] * 2,
            [pltpu.SemaphoreType.DMA] * 2,
        ],
    )
    def gather_kernel(xt_hbm, t4_hbm, out_hbm, idx_v, line_v, off_v,
                      gath_v, block_v, gsems, osems):
        wid = lax.axis_index("s") * nc + lax.axis_index("c")
        b0 = wid * bw
        pltpu.sync_copy(xt_hbm.at[:, pl.ds(b0, bw)], idx_v.at[pl.ds(0, S)])

        def prep_and_fire(s, buf):
            def per_group(g, _):
                iv = idx_v[s, pl.ds(g * L, L)]
                line_v[buf, pl.ds(g * L, L)] = lax.shift_right_logical(iv, 2)
                off_v[buf, pl.ds(g * L, L)] = (iv & (pack - 1)) * D
                return _

            lax.fori_loop(0, ng, per_group, None)
            pltpu.async_copy(
                t4_hbm.at[line_v.at[buf]], gath_v.at[buf], gsems[buf]
            )

        def wait_gather(buf):
            pltpu.make_async_copy(
                t4_hbm.at[pl.ds(0, bw), :], gath_v.at[buf], gsems[buf]
            ).wait()

        def transpose_block(buf):
            rows = lax.broadcasted_iota(jnp.int32, (L,), 0)

            def per_cell(t, _):
                d = t // ng
                g = lax.rem(t, ng)
                offs = off_v[buf, pl.ds(g * L, L)]
                vals = plsc.load_gather(
                    gath_v.at[buf], [rows + g * L, offs + d]
                )
                block_v[buf, d, pl.ds(g * L, L)] = vals
                return _

            lax.fori_loop(0, D * ng, per_cell, None)

        def store_block(s, buf):
            pltpu.async_copy(
                block_v.at[buf], out_hbm.at[s, :, pl.ds(b0, bw)], osems[buf]
            )

        def wait_store(buf):
            pltpu.make_async_copy(
                block_v.at[buf], out_hbm.at[0, :, pl.ds(b0, bw)], osems[buf]
            ).wait()

        prep_and_fire(0, 0)

        def step(s, buf, nbuf):
            @pl.when(s + 1 < S)
            def _fire_next():
                prep_and_fire(s + 1, nbuf)

            wait_gather(buf)

            @pl.when(s >= 2)
            def _drain_store():
                wait_store(buf)

            transpose_block(buf)
            store_block(s, buf)

        def per_pair(k, _):
            step(2 * k, 0, 1)
            step(2 * k + 1, 1, 0)
            return _

        lax.fori_loop(0, S // 2, per_pair, None)
        wait_store(0)
        wait_store(1)

    return gather_kernel


def kernel(x, table):
    B0, S = x.shape
    V, D = table.shape
    t4 = table.reshape(V * D // 128, 128)
    outT = _make_gather(S, B0, V, D)(x.T, t4)
    return outT.transpose(2, 0, 1)
